# Initial kernel scaffold; baseline (speedup 1.0000x reference)
import jax
import jax.numpy as jnp
from jax.experimental import pallas as pl


def kernel(output, mask, ind, target):
    raise NotImplementedError("write your pallas kernel here")


# speedup vs baseline: 3.3590x; 3.3590x over previous
import functools

import jax
import jax.numpy as jnp
from jax import lax
from jax.experimental import pallas as pl
from jax.experimental.pallas import tpu as pltpu
from jax.experimental.pallas import tpu_sc as plsc

_NC, _NS, _L = 2, 16, 16
_NW = _NC * _NS

_B, _C, _H, _W = 8, 256, 128, 128
_K = 128
_HW = _H * _W
_NP = _B * _K
_PPW = _NP // _NW
_NCHUNK = _PPW // _L

_P = (-6.07487764e-03, 3.44185935e-02, -9.23137686e-02, 1.64783497e-01,
      -2.39190717e-01, 3.31334006e-01, -4.99801163e-01, 9.99991455e-01,
      9.08378818e-08)


def _softplus_neg(z):
    e = jnp.exp(z)
    acc = jnp.full_like(e, _P[0])
    for c in _P[1:]:
        acc = acc * e + c
    return acc


_CSUB = 64
_NSUB_PER_CHUNK = _C // _CSUB
_NSUB = _NCHUNK * _NSUB_PER_CHUNK
_EPS = _L * _CSUB
_EPC = _L * _C


def _sc_body(table_hbm, ind_hbm, mask_hbm, tgt_hbm, out_hbm,
             ind_v, mask_v, tgt_v, idx_v, rows0, rows1, out_v,
             sem0, sem1, sem_in):
    wid = lax.axis_index("s") * _NC + lax.axis_index("c")
    base_pair = wid * _PPW
    pltpu.sync_copy(ind_hbm.at[pl.ds(base_pair, _PPW)], ind_v)
    pltpu.sync_copy(mask_hbm.at[pl.ds(base_pair, _PPW)], mask_v)
    tgt_cp = pltpu.async_copy(
        tgt_hbm.at[pl.ds(base_pair * _C, _PPW * _C)], tgt_v, sem_in)

    iot = lax.iota(jnp.int32, _L)
    rings = (rows0, rows1)
    sems = (sem0, sem1)

    def _build_chunk(chunk):
        iv = ind_v[pl.ds(chunk * _L, _L)]
        q = base_pair + chunk * _L + iot
        b = lax.shift_right_logical(q, 7)
        base = b * (_C * _HW) + iv

        def _build(c, vec, chunk=chunk):
            idx_v[pl.ds(chunk * _EPC + c * _L, _L)] = vec
            return vec + _HW
        lax.fori_loop(0, _C, _build, base)

    def _fire(s):
        return [
            pltpu.async_copy(
                table_hbm.at[idx_v.at[pl.ds(s * _EPS + j * 128, 128)]],
                rings[s % 2].at[pl.ds(j * 128, 128)], sems[s % 2])
            for j in range(_EPS // 128)
        ]

    def _compute(s, acc):
        chunk, cblk = divmod(s, _NSUB_PER_CHUNK)
        c0 = cblk * _CSUB

        def _accum(i, carry, chunk=chunk, c0=c0, s=s):
            pred = rings[s % 2][pl.ds(i * _L, _L)]
            tgt = tgt_v[pl.ds((c0 + i) * _PPW + chunk * _L, _L)]
            bce = (jnp.maximum(pred, 0.0) - pred * tgt
                   + _softplus_neg(-jnp.abs(pred)))
            return carry + bce
        return lax.fori_loop(0, _CSUB, _accum, acc)

    _build_chunk(0)
    cps = _fire(0)
    _build_chunk(1)
    tgt_cp.wait()

    accs = [jnp.zeros((_L,), jnp.float32) for _ in range(_NCHUNK)]
    for s in range(_NSUB):
        nxt = _fire(s + 1) if s + 1 < _NSUB else []
        for cp in cps:
            cp.wait()
        accs[s // _NSUB_PER_CHUNK] = _compute(s, accs[s // _NSUB_PER_CHUNK])
        cps = nxt

    loss_vec = jnp.zeros((_L,), jnp.float32)
    cnt_vec = jnp.zeros((_L,), jnp.float32)
    for chunk in range(_NCHUNK):
        mf = mask_v[pl.ds(chunk * _L, _L)].astype(jnp.float32)
        loss_vec = loss_vec + accs[chunk] * mf
        cnt_vec = cnt_vec + mf

    out_v[pl.ds(0, _L)] = loss_vec
    out_v[pl.ds(_L, _L)] = cnt_vec
    pltpu.sync_copy(out_v, out_hbm.at[wid])


@jax.jit
def kernel(output, mask, ind, target):
    table = output.reshape(_B * _C * _HW)
    ind_flat = ind.reshape(_NP).astype(jnp.int32)
    mask_flat = mask.reshape(_NP).astype(jnp.int32)
    tgt = (target.reshape(_NW, _PPW, _C)
           .transpose(0, 2, 1).reshape(_NW * _C * _PPW))

    call = pl.kernel(
        _sc_body,
        out_type=jax.ShapeDtypeStruct((_NW, 2 * _L), jnp.float32),
        mesh=plsc.VectorSubcoreMesh(core_axis_name="c", subcore_axis_name="s",
                                    num_cores=_NC, num_subcores=_NS),
        scratch_types=[
            pltpu.VMEM((_PPW,), jnp.int32),
            pltpu.VMEM((_PPW,), jnp.int32),
            pltpu.VMEM((_C * _PPW,), jnp.float32),
            pltpu.VMEM((_NCHUNK * _EPC,), jnp.int32),
            pltpu.VMEM((_EPS,), jnp.float32),
            pltpu.VMEM((_EPS,), jnp.float32),
            pltpu.VMEM((2 * _L,), jnp.float32),
            pltpu.SemaphoreType.DMA,
            pltpu.SemaphoreType.DMA,
            pltpu.SemaphoreType.DMA,
        ],
    )
    partials = call(table, ind_flat, mask_flat, tgt)
    loss_sum = jnp.sum(partials[:, :_L])
    denom = jnp.sum(partials[:, _L:]) * _C + 0.0001
    return loss_sum / denom
